# initial kernel scaffold (unmeasured)
import functools

import jax
import jax.numpy as jnp
from jax import lax
from jax.experimental import pallas as pl
from jax.experimental.pallas import tpu as pltpu

N_DEV = 4
E_PER_DEV = 4
N_TOK = 1024
D_MODEL = 512
D_FF = 1024
ROWS_PER_DEV = N_TOK // N_DEV


def kernel(x, router_W, route_idx, expert_W, shared_W):
    def body(x_ref, rw_ref, idx_ref, ew_ref, sw_ref, out_ref,
             partial_ref, recv_ref, send_sems, recv_sems):
        my_pos = lax.axis_index("i")
        left = (my_pos - 1) % N_DEV
        right = (my_pos + 1) % N_DEV

        barrier_sem = pltpu.get_barrier_semaphore()
        for nbr in [left, right]:
            pl.semaphore_signal(
                barrier_sem, inc=1,
                device_id=(nbr,), device_id_type=pl.DeviceIdType.MESH,
            )
        pl.semaphore_wait(barrier_sem, 2)

        xf = x_ref[:, :]
        scores = jnp.dot(xf, rw_ref[:, :], preferred_element_type=jnp.float32)
        scores = scores - jnp.max(scores, axis=-1, keepdims=True)
        e_scores = jnp.exp(scores)
        probs = e_scores / jnp.sum(e_scores, axis=-1, keepdims=True)
        idx = idx_ref[:, :]
        lanes = lax.broadcasted_iota(jnp.int32, (N_TOK, 16), 1)
        sel_prob = jnp.sum(
            jnp.where(lanes == idx, probs, 0.0), axis=-1, keepdims=True
        )

        partial = jnp.zeros((N_TOK, D_FF), jnp.float32)
        for e_local in range(E_PER_DEV):
            e_glob = my_pos * E_PER_DEV + e_local
            coeff = jnp.where(idx == e_glob, sel_prob, 0.0)
            xm = (xf * coeff).astype(jnp.bfloat16)
            partial = partial + jnp.dot(
                xm, ew_ref[e_local, :, :].astype(jnp.bfloat16),
                preferred_element_type=jnp.float32,
            )
        partial_ref[:, :] = partial

        for s in range(N_DEV - 1):
            send_chunk = (my_pos + (N_DEV - 1) - s) % N_DEV
            rdma = pltpu.make_async_remote_copy(
                src_ref=partial_ref.at[pl.ds(send_chunk * ROWS_PER_DEV,
                                             ROWS_PER_DEV)],
                dst_ref=recv_ref.at[s],
                send_sem=send_sems.at[s],
                recv_sem=recv_sems.at[s],
                device_id=(right,),
                device_id_type=pl.DeviceIdType.MESH,
            )
            rdma.start()
            rdma.wait()
            recv_chunk = (my_pos + (N_DEV - 2) - s) % N_DEV
            rows = pl.ds(recv_chunk * ROWS_PER_DEV, ROWS_PER_DEV)
            partial_ref[rows, :] = partial_ref[rows, :] + recv_ref[s, :, :]

        my_rows = pl.ds(my_pos * ROWS_PER_DEV, ROWS_PER_DEV)
        shared = jnp.dot(
            xf[...] .astype(jnp.bfloat16)[my_pos * 0:, :][pl.ds(0, N_TOK), :],
            sw_ref[:, :].astype(jnp.bfloat16),
            preferred_element_type=jnp.float32,
        )
        out_ref[:, :] = partial_ref[my_rows, :] + shared[my_rows, :]

    return pl.pallas_call(
        body,
        out_shape=jax.ShapeDtypeStruct((ROWS_PER_DEV, D_FF), jnp.float32),
        in_specs=[pl.BlockSpec(memory_space=pltpu.VMEM)] * 5,
        out_specs=pl.BlockSpec(memory_space=pltpu.VMEM),
        scratch_shapes=[
            pltpu.VMEM((N_TOK, D_FF), jnp.float32),
            pltpu.VMEM((N_DEV - 1, ROWS_PER_DEV, D_FF), jnp.float32),
            pltpu.SemaphoreType.DMA((N_DEV - 1,)),
            pltpu.SemaphoreType.DMA((N_DEV - 1,)),
        ],
        compiler_params=pltpu.CompilerParams(collective_id=0),
    )(x, router_W, route_idx, expert_W, shared_W)


# baseline (device time: 58567 ns/iter reference)
import functools

import jax
import jax.numpy as jnp
from jax import lax
from jax.experimental import pallas as pl
from jax.experimental.pallas import tpu as pltpu

N_DEV = 4
E_PER_DEV = 4
N_TOK = 1024
D_MODEL = 512
D_FF = 1024
ROWS_PER_DEV = N_TOK // N_DEV


def kernel(x, router_W, route_idx, expert_W, shared_W):
    def body(x_ref, rw_ref, idx_ref, ew_ref, sw_ref, out_ref,
             partial_ref, recv_ref, send_sems, recv_sems):
        my_pos = lax.axis_index("i")
        left = (my_pos - 1) % N_DEV
        right = (my_pos + 1) % N_DEV

        barrier_sem = pltpu.get_barrier_semaphore()
        for nbr in [left, right]:
            pl.semaphore_signal(
                barrier_sem, inc=1,
                device_id=(nbr,), device_id_type=pl.DeviceIdType.MESH,
            )
        pl.semaphore_wait(barrier_sem, 2)

        xf = x_ref[:, :]
        scores = jnp.dot(xf, rw_ref[:, :], preferred_element_type=jnp.float32)
        scores = scores - jnp.max(scores, axis=-1, keepdims=True)
        e_scores = jnp.exp(scores)
        probs = e_scores / jnp.sum(e_scores, axis=-1, keepdims=True)
        idx = idx_ref[:, :]
        lanes = lax.broadcasted_iota(jnp.int32, (N_TOK, 16), 1)
        sel_prob = jnp.sum(
            jnp.where(lanes == idx, probs, 0.0), axis=-1, keepdims=True
        )

        partial = jnp.zeros((N_TOK, D_FF), jnp.float32)
        for e_local in range(E_PER_DEV):
            e_glob = my_pos * E_PER_DEV + e_local
            coeff = jnp.where(idx == e_glob, sel_prob, 0.0)
            xm = (xf * coeff).astype(jnp.bfloat16)
            partial = partial + jnp.dot(
                xm, ew_ref[e_local, :, :].astype(jnp.bfloat16),
                preferred_element_type=jnp.float32,
            )
        partial_ref[:, :] = partial

        for s in range(N_DEV - 1):
            send_chunk = (my_pos + (N_DEV - 1) - s) % N_DEV
            rdma = pltpu.make_async_remote_copy(
                src_ref=partial_ref.at[pl.ds(send_chunk * ROWS_PER_DEV,
                                             ROWS_PER_DEV)],
                dst_ref=recv_ref.at[s],
                send_sem=send_sems.at[s],
                recv_sem=recv_sems.at[s],
                device_id=(right,),
                device_id_type=pl.DeviceIdType.MESH,
            )
            rdma.start()
            rdma.wait()
            recv_chunk = (my_pos + (N_DEV - 2) - s) % N_DEV
            rows = pl.ds(recv_chunk * ROWS_PER_DEV, ROWS_PER_DEV)
            partial_ref[rows, :] = partial_ref[rows, :] + recv_ref[s, :, :]

        my_rows = pl.ds(my_pos * ROWS_PER_DEV, ROWS_PER_DEV)
        x_my = x_ref[my_rows, :].astype(jnp.bfloat16)
        shared = jnp.dot(
            x_my, sw_ref[:, :].astype(jnp.bfloat16),
            preferred_element_type=jnp.float32,
        )
        out_ref[:, :] = partial_ref[my_rows, :] + shared

    return pl.pallas_call(
        body,
        out_shape=jax.ShapeDtypeStruct((ROWS_PER_DEV, D_FF), jnp.float32),
        in_specs=[pl.BlockSpec(memory_space=pltpu.VMEM)] * 5,
        out_specs=pl.BlockSpec(memory_space=pltpu.VMEM),
        scratch_shapes=[
            pltpu.VMEM((N_TOK, D_FF), jnp.float32),
            pltpu.VMEM((N_DEV - 1, ROWS_PER_DEV, D_FF), jnp.float32),
            pltpu.SemaphoreType.DMA((N_DEV - 1,)),
            pltpu.SemaphoreType.DMA((N_DEV - 1,)),
        ],
        compiler_params=pltpu.CompilerParams(collective_id=0),
    )(x, router_W, route_idx, expert_W, shared_W)


# device time: 27998 ns/iter; 2.0918x vs baseline; 2.0918x over previous
import jax
import jax.numpy as jnp
from jax import lax
from jax.experimental import pallas as pl
from jax.experimental.pallas import tpu as pltpu

N_DEV = 4
E_PER_DEV = 4
N_TOK = 1024
D_MODEL = 512
D_FF = 1024
N_EXP = 16
ROWS = N_TOK // N_DEV


def kernel(x, router_W, route_idx, expert_W, shared_W):
    def body(x_ref, rw_ref, idx_ref, ew_ref, sw_ref, out_ref,
             send_ref, recv_ref, send_sems, recv_sems):
        my_pos = lax.axis_index("i")

        barrier_sem = pltpu.get_barrier_semaphore()
        for off in range(1, N_DEV):
            pl.semaphore_signal(
                barrier_sem, inc=1,
                device_id=((my_pos + off) % N_DEV,),
                device_id_type=pl.DeviceIdType.MESH,
            )
        pl.semaphore_wait(barrier_sem, N_DEV - 1)

        def chunk_partial(c, acc_dtype):
            rows = pl.ds(c * ROWS, ROWS)
            xc = x_ref[rows, :]
            scores = jnp.dot(xc, rw_ref[:, :],
                             preferred_element_type=jnp.float32)
            scores = scores - jnp.max(scores, axis=-1, keepdims=True)
            es = jnp.exp(scores)
            probs = es / jnp.sum(es, axis=-1, keepdims=True)
            idx = idx_ref[rows, :]
            lanes = lax.broadcasted_iota(jnp.int32, (ROWS, N_EXP), 1)
            sel_prob = jnp.sum(jnp.where(lanes == idx, probs, 0.0),
                               axis=-1, keepdims=True)
            part = jnp.zeros((ROWS, D_FF), jnp.float32)
            for e_local in range(E_PER_DEV):
                e_glob = my_pos * E_PER_DEV + e_local
                coeff = jnp.where(idx == e_glob, sel_prob, 0.0)
                xm = (xc * coeff).astype(jnp.bfloat16)
                part = part + jnp.dot(
                    xm, ew_ref[e_local, :, :].astype(jnp.bfloat16),
                    preferred_element_type=jnp.float32,
                )
            return part.astype(acc_dtype)

        rdmas = []
        for k in (1, 0, 2):
            peer = (my_pos + 1 + k) % N_DEV
            send_ref[k, :, :] = chunk_partial(peer, jnp.bfloat16)
            rdma = pltpu.make_async_remote_copy(
                src_ref=send_ref.at[k],
                dst_ref=recv_ref.at[2 - k],
                send_sem=send_sems.at[k],
                recv_sem=recv_sems.at[2 - k],
                device_id=(peer,),
                device_id_type=pl.DeviceIdType.MESH,
            )
            rdma.start()
            rdmas.append(rdma)

        mine = chunk_partial(my_pos, jnp.float32)
        my_rows = pl.ds(my_pos * ROWS, ROWS)
        shared = jnp.dot(
            x_ref[my_rows, :].astype(jnp.bfloat16),
            sw_ref[:, :].astype(jnp.bfloat16),
            preferred_element_type=jnp.float32,
        )

        for rdma in rdmas:
            rdma.wait_recv()
        acc = mine + shared
        for j in range(N_DEV - 1):
            acc = acc + recv_ref[j, :, :].astype(jnp.float32)
        out_ref[:, :] = acc
        for rdma in rdmas:
            rdma.wait_send()

    return pl.pallas_call(
        body,
        out_shape=jax.ShapeDtypeStruct((ROWS, D_FF), jnp.float32),
        in_specs=[pl.BlockSpec(memory_space=pltpu.VMEM)] * 5,
        out_specs=pl.BlockSpec(memory_space=pltpu.VMEM),
        scratch_shapes=[
            pltpu.VMEM((N_DEV - 1, ROWS, D_FF), jnp.bfloat16),
            pltpu.VMEM((N_DEV - 1, ROWS, D_FF), jnp.bfloat16),
            pltpu.SemaphoreType.DMA((N_DEV - 1,)),
            pltpu.SemaphoreType.DMA((N_DEV - 1,)),
        ],
        compiler_params=pltpu.CompilerParams(collective_id=0),
    )(x, router_W, route_idx, expert_W, shared_W)
